# baseline (speedup 1.0000x reference)
import functools

import jax
import jax.numpy as jnp
from jax import lax
from jax.experimental import pallas as pl
from jax.experimental.pallas import tpu as pltpu
from jax.experimental.pallas import tpu_sc as plsc

N_NODES = 50000
N_EDGES = 800000
IN_CH = 128
HID = 64
HALF = HID // 2
N_GRAPHS = 64

NC = 2
NS = 16
CHUNK = 128

E_PAD = 819200
N_CHUNK_ROWS = E_PAD // CHUNK
ACC_ROWS = 50176
ROWS_PT = ACC_ROWS // NS
TRASH = N_NODES

IDXR = 2
BCHUNK = IDXR * CHUNK
CPT_AGG = E_PAD // NS // BCHUNK
STG_AGG = 25
SPC_AGG = CPT_AGG // STG_AGG
CPT_DEG = E_PAD // (NC * NS) // CHUNK
STG_DEG = 5
SPC_DEG = CPT_DEG // STG_DEG

ROW_BLK = 1000
N_BLK = N_NODES // ROW_BLK

_sc_mesh = plsc.VectorSubcoreMesh(
    core_axis_name="c", subcore_axis_name="s", num_cores=NC, num_subcores=NS)


def _deg_body(col2, zeros16, ones16, out, col_st, ones_v, acc):
    c = lax.axis_index("c")
    s = lax.axis_index("s")
    pltpu.sync_copy(zeros16, acc.at[pl.ds(s * ROWS_PT, ROWS_PT)])
    pltpu.sync_copy(ones16, ones_v)
    plsc.subcore_barrier()
    base = (c * NS + s) * CPT_DEG

    def stage(st, carry):
        crb = base + st * SPC_DEG
        pltpu.sync_copy(col2.at[pl.ds(crb, SPC_DEG)], col_st)
        for j in range(SPC_DEG):
            pltpu.sync_copy(ones_v, acc.at[col_st.at[j]], add=True)
        return carry

    lax.fori_loop(0, STG_DEG, stage, 0)
    plsc.subcore_barrier()
    pltpu.sync_copy(acc.at[pl.ds(s * ROWS_PT, ROWS_PT)], out.at[c, s])


_deg_call = pl.kernel(
    _deg_body,
    out_type=jax.ShapeDtypeStruct((NC, NS, ROWS_PT, 16), jnp.float32),
    mesh=_sc_mesh,
    scratch_types=[
        pltpu.VMEM((SPC_DEG, CHUNK), jnp.int32),
        pltpu.VMEM((CHUNK, 16), jnp.float32),
        pltpu.VMEM_SHARED((ACC_ROWS, 16), jnp.float32),
    ],
    compiler_params=pltpu.CompilerParams(use_tc_tiling_on_sc=False),
)


NBUF = 3
KLAG = 1


def _agg_body(g2, rowb, col3, zeros32, out, row_st, col_st, msg, acc, gsems,
              ssems):
    c = lax.axis_index("c")
    s = lax.axis_index("s")
    pltpu.sync_copy(zeros32, acc.at[pl.ds(s * ROWS_PT, ROWS_PT)])
    plsc.subcore_barrier()
    base = s * CPT_AGG

    def stage(st, carry):
        crb = base + st * SPC_AGG
        pltpu.sync_copy(rowb.at[c, pl.ds(crb, SPC_AGG)], row_st)
        pltpu.sync_copy(col3.at[pl.ds(crb, SPC_AGG)], col_st)
        for j in range(SPC_AGG + KLAG):
            if j < SPC_AGG:
                b = j % NBUF
                if j >= NBUF:
                    pltpu.make_async_copy(msg.at[b],
                                          acc.at[col_st.at[j - NBUF]],
                                          ssems.at[b]).wait()
                pltpu.async_copy(g2.at[row_st.at[j]], msg.at[b], gsems.at[b])
            i = j - KLAG
            if i >= 0:
                bi = i % NBUF
                pltpu.make_async_copy(g2.at[row_st.at[i]], msg.at[bi],
                                      gsems.at[bi]).wait()
                pltpu.async_copy(msg.at[bi], acc.at[col_st.at[i]],
                                 ssems.at[bi], add=True)
        for k in range(NBUF):
            i = SPC_AGG - NBUF + k
            pltpu.make_async_copy(msg.at[i % NBUF], acc.at[col_st.at[i]],
                                  ssems.at[i % NBUF]).wait()
        return carry

    lax.fori_loop(0, STG_AGG, stage, 0)
    plsc.subcore_barrier()
    pltpu.sync_copy(acc.at[pl.ds(s * ROWS_PT, ROWS_PT)], out.at[c, s])


_agg_call = pl.kernel(
    _agg_body,
    out_type=jax.ShapeDtypeStruct((NC, NS, ROWS_PT, HALF), jnp.float32),
    mesh=_sc_mesh,
    scratch_types=[
        pltpu.VMEM((SPC_AGG, BCHUNK), jnp.int32),
        pltpu.VMEM((SPC_AGG, BCHUNK), jnp.int32),
        pltpu.VMEM((NBUF, BCHUNK, HALF), jnp.float32),
        pltpu.VMEM_SHARED((ACC_ROWS, HALF), jnp.float32),
        pltpu.SemaphoreType.DMA((NBUF,)),
        pltpu.SemaphoreType.DMA((NBUF,)),
    ],
    compiler_params=pltpu.CompilerParams(use_tc_tiling_on_sc=False),
)


def _enc_body(degp_ref, x_ref, w_ref, dis_ref, g_ref):
    deg = degp_ref[0, :, 0] + degp_ref[1, :, 0] + 1.0
    dis = lax.rsqrt(deg)
    dis_ref[...] = dis[:, None]
    g = jnp.dot(x_ref[...], w_ref[...], preferred_element_type=jnp.float32)
    g_ref[...] = g * dis[:, None]


def _enc_call(degp, x, w1):
    return pl.pallas_call(
        _enc_body,
        grid=(N_BLK,),
        in_specs=[
            pl.BlockSpec((NC, ROW_BLK, 16), lambda i: (0, i, 0)),
            pl.BlockSpec((ROW_BLK, IN_CH), lambda i: (i, 0)),
            pl.BlockSpec((IN_CH, HID), lambda i: (0, 0)),
        ],
        out_specs=[
            pl.BlockSpec((ROW_BLK, 1), lambda i: (i, 0)),
            pl.BlockSpec((ROW_BLK, HID), lambda i: (i, 0)),
        ],
        out_shape=[
            jax.ShapeDtypeStruct((N_NODES, 1), jnp.float32),
            jax.ShapeDtypeStruct((N_NODES, HID), jnp.float32),
        ],
    )(degp, x, w1)


def _layer_body(aggp_ref, g_ref, dis_ref, w_ref, b_ref, gn_ref):
    agg = jnp.concatenate([aggp_ref[0], aggp_ref[1]], axis=1)
    dis = dis_ref[...]
    h = jnp.maximum(dis * (agg + g_ref[...]) + b_ref[...], 0.0)
    gn_ref[...] = jnp.dot(h, w_ref[...], preferred_element_type=jnp.float32) * dis


def _layer_call(aggp, g, dis, w, b):
    return pl.pallas_call(
        _layer_body,
        grid=(N_BLK,),
        in_specs=[
            pl.BlockSpec((NC, ROW_BLK, HALF), lambda i: (0, i, 0)),
            pl.BlockSpec((ROW_BLK, HID), lambda i: (i, 0)),
            pl.BlockSpec((ROW_BLK, 1), lambda i: (i, 0)),
            pl.BlockSpec((HID, HID), lambda i: (0, 0)),
            pl.BlockSpec((1, HID), lambda i: (0, 0)),
        ],
        out_specs=pl.BlockSpec((ROW_BLK, HID), lambda i: (i, 0)),
        out_shape=jax.ShapeDtypeStruct((N_NODES, HID), jnp.float32),
    )(aggp, g, dis, w, b)


def _pool_body(aggp_ref, g_ref, dis_ref, b_ref, batch_ref, sums_ref, cnt_ref):
    i = pl.program_id(0)
    agg = jnp.concatenate([aggp_ref[0], aggp_ref[1]], axis=1)
    dis = dis_ref[...]
    h = jnp.maximum(dis * (agg + g_ref[...]) + b_ref[...], 0.0)
    bv = batch_ref[...][:, 0]
    gids = lax.broadcasted_iota(jnp.int32, (N_GRAPHS, ROW_BLK), 0)
    m = (gids == bv[None, :]).astype(jnp.float32)
    ps = jnp.dot(m, h, preferred_element_type=jnp.float32)
    pc = jnp.sum(m, axis=1, keepdims=True)

    @pl.when(i == 0)
    def _():
        sums_ref[...] = ps
        cnt_ref[...] = pc

    @pl.when(i > 0)
    def _():
        sums_ref[...] += ps
        cnt_ref[...] += pc


def _pool_call(aggp, g, dis, b, batch2):
    return pl.pallas_call(
        _pool_body,
        grid=(N_BLK,),
        in_specs=[
            pl.BlockSpec((NC, ROW_BLK, HALF), lambda i: (0, i, 0)),
            pl.BlockSpec((ROW_BLK, HID), lambda i: (i, 0)),
            pl.BlockSpec((ROW_BLK, 1), lambda i: (i, 0)),
            pl.BlockSpec((1, HID), lambda i: (0, 0)),
            pl.BlockSpec((ROW_BLK, 1), lambda i: (i, 0)),
        ],
        out_specs=[
            pl.BlockSpec((N_GRAPHS, HID), lambda i: (0, 0)),
            pl.BlockSpec((N_GRAPHS, 1), lambda i: (0, 0)),
        ],
        out_shape=[
            jax.ShapeDtypeStruct((N_GRAPHS, HID), jnp.float32),
            jax.ShapeDtypeStruct((N_GRAPHS, 1), jnp.float32),
        ],
    )(aggp, g, dis, b, batch2)


def _head_body(sums_ref, cnt_ref, meta_ref, wa_ref, wb_ref, bh1_ref, wh2_ref,
               bh2_ref, out_ref):
    pooled = sums_ref[...] / jnp.maximum(cnt_ref[...], 1.0)
    z1 = (jnp.dot(pooled, wa_ref[...], preferred_element_type=jnp.float32)
          + jnp.dot(meta_ref[...], wb_ref[...], preferred_element_type=jnp.float32)
          + bh1_ref[...])
    z1 = jnp.maximum(z1, 0.0)
    out_ref[...] = jnp.dot(z1, wh2_ref[...], preferred_element_type=jnp.float32) + bh2_ref[...]


def _head_call(sums, cnt, meta, wa, wb, bh1, wh2, bh2):
    return pl.pallas_call(
        _head_body,
        out_shape=jax.ShapeDtypeStruct((N_GRAPHS, 1), jnp.float32),
    )(sums, cnt, meta, wa, wb, bh1, wh2, bh2)


def kernel(x, edge_index, batch, metadata, W1, b1, W2, b2, W3, b3, Wh1, bh1,
           Wh2, bh2):
    pad = E_PAD - N_EDGES
    rowp = jnp.concatenate(
        [edge_index[0], jnp.zeros((pad,), edge_index.dtype)]).reshape(N_CHUNK_ROWS, CHUNK)
    colp = jnp.concatenate(
        [edge_index[1], jnp.full((pad,), TRASH, edge_index.dtype)]).reshape(N_CHUNK_ROWS, CHUNK)
    rowp = rowp.astype(jnp.int32)
    colp = colp.astype(jnp.int32)
    rowb = jnp.stack([rowp * 2, rowp * 2 + 1]).reshape(
        NC, N_CHUNK_ROWS // IDXR, BCHUNK)
    colp3 = colp.reshape(N_CHUNK_ROWS // IDXR, BCHUNK)
    zeros16 = jnp.zeros((ROWS_PT, 16), jnp.float32)
    zeros32 = jnp.zeros((ROWS_PT, HALF), jnp.float32)
    ones16 = jnp.ones((CHUNK, 16), jnp.float32)

    degp = _deg_call(colp, zeros16, ones16).reshape(NC, ACC_ROWS, 16)
    dis, g1 = _enc_call(degp, x, W1)

    def agg(g):
        out = _agg_call(g.reshape(2 * N_NODES, HALF), rowb, colp3, zeros32)
        return out.reshape(NC, ACC_ROWS, HALF)

    b1r = b1.reshape(1, HID)
    b2r = b2.reshape(1, HID)
    b3r = b3.reshape(1, HID)

    agg1 = agg(g1)
    g2 = _layer_call(agg1, g1, dis, W2, b1r)
    agg2 = agg(g2)
    g3 = _layer_call(agg2, g2, dis, W3, b2r)
    agg3 = agg(g3)

    sums, cnt = _pool_call(agg3, g3, dis, b3r, batch.reshape(N_NODES, 1).astype(jnp.int32))
    out = _head_call(sums, cnt, metadata, Wh1[:HID], Wh1[HID:],
                     bh1.reshape(1, HID), Wh2, bh2.reshape(1, 1))
    return out


# speedup vs baseline: 1.3979x; 1.3979x over previous
import functools

import jax
import jax.numpy as jnp
from jax import lax
from jax.experimental import pallas as pl
from jax.experimental.pallas import tpu as pltpu
from jax.experimental.pallas import tpu_sc as plsc

N_NODES = 50000
N_EDGES = 800000
IN_CH = 128
HID = 64
HALF = HID // 2
N_GRAPHS = 64

NC = 2
NS = 16
CHUNK = 128

E_PAD = 819200
N_CHUNK_ROWS = E_PAD // CHUNK
ACC_ROWS = 50176
ROWS_PT = ACC_ROWS // NS
TRASH = N_NODES

IDXR = 2
BCHUNK = IDXR * CHUNK
CPT_AGG = E_PAD // NS // BCHUNK
STG_AGG = 25
SPC_AGG = CPT_AGG // STG_AGG
CPT_DEG = E_PAD // (NC * NS) // CHUNK
STG_DEG = 5
SPC_DEG = CPT_DEG // STG_DEG

ROW_BLK = 1000
N_BLK = N_NODES // ROW_BLK

_sc_mesh = plsc.VectorSubcoreMesh(
    core_axis_name="c", subcore_axis_name="s", num_cores=NC, num_subcores=NS)


def _deg_body(col2, zeros16, ones16, out, col_st, ones_v, acc):
    c = lax.axis_index("c")
    s = lax.axis_index("s")
    pltpu.sync_copy(zeros16, acc.at[pl.ds(s * ROWS_PT, ROWS_PT)])
    pltpu.sync_copy(ones16, ones_v)
    plsc.subcore_barrier()
    base = (c * NS + s) * CPT_DEG

    def stage(st, carry):
        crb = base + st * SPC_DEG
        pltpu.sync_copy(col2.at[pl.ds(crb, SPC_DEG)], col_st)
        for j in range(SPC_DEG):
            pltpu.sync_copy(ones_v, acc.at[col_st.at[j]], add=True)
        return carry

    lax.fori_loop(0, STG_DEG, stage, 0)
    plsc.subcore_barrier()
    pltpu.sync_copy(acc.at[pl.ds(s * ROWS_PT, ROWS_PT)], out.at[c, s])


_deg_call = pl.kernel(
    _deg_body,
    out_type=jax.ShapeDtypeStruct((NC, NS, ROWS_PT, 16), jnp.float32),
    mesh=_sc_mesh,
    scratch_types=[
        pltpu.VMEM((SPC_DEG, CHUNK), jnp.int32),
        pltpu.VMEM((CHUNK, 16), jnp.float32),
        pltpu.VMEM_SHARED((ACC_ROWS, 16), jnp.float32),
    ],
    compiler_params=pltpu.CompilerParams(use_tc_tiling_on_sc=False),
)


NBUF = 6
KLAG = 3


def _agg_body(g2, rowb, col3, zeros32, out, row_st, col_st, msg, acc, gsems,
              ssems):
    c = lax.axis_index("c")
    s = lax.axis_index("s")
    pltpu.sync_copy(zeros32, acc.at[pl.ds(s * ROWS_PT, ROWS_PT)])
    plsc.subcore_barrier()
    base = s * CPT_AGG

    def stage(st, carry):
        crb = base + st * SPC_AGG
        pltpu.sync_copy(rowb.at[c, pl.ds(crb, SPC_AGG)], row_st)
        pltpu.sync_copy(col3.at[pl.ds(crb, SPC_AGG)], col_st)
        for j in range(SPC_AGG + KLAG):
            if j < SPC_AGG:
                b = j % NBUF
                if j >= NBUF:
                    pltpu.make_async_copy(msg.at[b],
                                          acc.at[col_st.at[j - NBUF]],
                                          ssems.at[b]).wait()
                pltpu.async_copy(g2.at[row_st.at[j]], msg.at[b], gsems.at[b])
            i = j - KLAG
            if i >= 0:
                bi = i % NBUF
                pltpu.make_async_copy(g2.at[row_st.at[i]], msg.at[bi],
                                      gsems.at[bi]).wait()
                pltpu.async_copy(msg.at[bi], acc.at[col_st.at[i]],
                                 ssems.at[bi], add=True)
        for k in range(NBUF):
            i = SPC_AGG - NBUF + k
            pltpu.make_async_copy(msg.at[i % NBUF], acc.at[col_st.at[i]],
                                  ssems.at[i % NBUF]).wait()
        return carry

    lax.fori_loop(0, STG_AGG, stage, 0)
    plsc.subcore_barrier()
    pltpu.sync_copy(acc.at[pl.ds(s * ROWS_PT, ROWS_PT)], out.at[c, s])


_agg_call = pl.kernel(
    _agg_body,
    out_type=jax.ShapeDtypeStruct((NC, NS, ROWS_PT, HALF), jnp.bfloat16),
    mesh=_sc_mesh,
    scratch_types=[
        pltpu.VMEM((SPC_AGG, BCHUNK), jnp.int32),
        pltpu.VMEM((SPC_AGG, BCHUNK), jnp.int32),
        pltpu.VMEM((NBUF, BCHUNK, HALF), jnp.bfloat16),
        pltpu.VMEM_SHARED((ACC_ROWS, HALF), jnp.bfloat16),
        pltpu.SemaphoreType.DMA((NBUF,)),
        pltpu.SemaphoreType.DMA((NBUF,)),
    ],
    compiler_params=pltpu.CompilerParams(use_tc_tiling_on_sc=False),
)


def _enc_body(degp_ref, x_ref, w_ref, dis_ref, g_ref, gb_ref):
    deg = degp_ref[0, :, 0] + degp_ref[1, :, 0] + 1.0
    dis = lax.rsqrt(deg)
    dis_ref[...] = dis[:, None]
    g = jnp.dot(x_ref[...], w_ref[...], preferred_element_type=jnp.float32)
    g = g * dis[:, None]
    g_ref[...] = g
    gb_ref[...] = g.astype(jnp.bfloat16)


def _enc_call(degp, x, w1):
    return pl.pallas_call(
        _enc_body,
        grid=(N_BLK,),
        in_specs=[
            pl.BlockSpec((NC, ROW_BLK, 16), lambda i: (0, i, 0)),
            pl.BlockSpec((ROW_BLK, IN_CH), lambda i: (i, 0)),
            pl.BlockSpec((IN_CH, HID), lambda i: (0, 0)),
        ],
        out_specs=[
            pl.BlockSpec((ROW_BLK, 1), lambda i: (i, 0)),
            pl.BlockSpec((ROW_BLK, HID), lambda i: (i, 0)),
            pl.BlockSpec((ROW_BLK, HID), lambda i: (i, 0)),
        ],
        out_shape=[
            jax.ShapeDtypeStruct((N_NODES, 1), jnp.float32),
            jax.ShapeDtypeStruct((N_NODES, HID), jnp.float32),
            jax.ShapeDtypeStruct((N_NODES, HID), jnp.bfloat16),
        ],
    )(degp, x, w1)


def _layer_body(aggp_ref, g_ref, dis_ref, w_ref, b_ref, gn_ref, gnb_ref):
    agg = jnp.concatenate([aggp_ref[0], aggp_ref[1]],
                          axis=1).astype(jnp.float32)
    dis = dis_ref[...]
    h = jnp.maximum(dis * (agg + g_ref[...]) + b_ref[...], 0.0)
    gn = jnp.dot(h, w_ref[...], preferred_element_type=jnp.float32) * dis
    gn_ref[...] = gn
    gnb_ref[...] = gn.astype(jnp.bfloat16)


def _layer_call(aggp, g, dis, w, b):
    return pl.pallas_call(
        _layer_body,
        grid=(N_BLK,),
        in_specs=[
            pl.BlockSpec((NC, ROW_BLK, HALF), lambda i: (0, i, 0)),
            pl.BlockSpec((ROW_BLK, HID), lambda i: (i, 0)),
            pl.BlockSpec((ROW_BLK, 1), lambda i: (i, 0)),
            pl.BlockSpec((HID, HID), lambda i: (0, 0)),
            pl.BlockSpec((1, HID), lambda i: (0, 0)),
        ],
        out_specs=[
            pl.BlockSpec((ROW_BLK, HID), lambda i: (i, 0)),
            pl.BlockSpec((ROW_BLK, HID), lambda i: (i, 0)),
        ],
        out_shape=[
            jax.ShapeDtypeStruct((N_NODES, HID), jnp.float32),
            jax.ShapeDtypeStruct((N_NODES, HID), jnp.bfloat16),
        ],
    )(aggp, g, dis, w, b)


def _pool_body(aggp_ref, g_ref, dis_ref, b_ref, batch_ref, sums_ref, cnt_ref):
    i = pl.program_id(0)
    agg = jnp.concatenate([aggp_ref[0], aggp_ref[1]],
                          axis=1).astype(jnp.float32)
    dis = dis_ref[...]
    h = jnp.maximum(dis * (agg + g_ref[...]) + b_ref[...], 0.0)
    bv = batch_ref[...][:, 0]
    gids = lax.broadcasted_iota(jnp.int32, (N_GRAPHS, ROW_BLK), 0)
    m = (gids == bv[None, :]).astype(jnp.float32)
    ps = jnp.dot(m, h, preferred_element_type=jnp.float32)
    pc = jnp.sum(m, axis=1, keepdims=True)

    @pl.when(i == 0)
    def _():
        sums_ref[...] = ps
        cnt_ref[...] = pc

    @pl.when(i > 0)
    def _():
        sums_ref[...] += ps
        cnt_ref[...] += pc


def _pool_call(aggp, g, dis, b, batch2):
    return pl.pallas_call(
        _pool_body,
        grid=(N_BLK,),
        in_specs=[
            pl.BlockSpec((NC, ROW_BLK, HALF), lambda i: (0, i, 0)),
            pl.BlockSpec((ROW_BLK, HID), lambda i: (i, 0)),
            pl.BlockSpec((ROW_BLK, 1), lambda i: (i, 0)),
            pl.BlockSpec((1, HID), lambda i: (0, 0)),
            pl.BlockSpec((ROW_BLK, 1), lambda i: (i, 0)),
        ],
        out_specs=[
            pl.BlockSpec((N_GRAPHS, HID), lambda i: (0, 0)),
            pl.BlockSpec((N_GRAPHS, 1), lambda i: (0, 0)),
        ],
        out_shape=[
            jax.ShapeDtypeStruct((N_GRAPHS, HID), jnp.float32),
            jax.ShapeDtypeStruct((N_GRAPHS, 1), jnp.float32),
        ],
    )(aggp, g, dis, b, batch2)


def _head_body(sums_ref, cnt_ref, meta_ref, wa_ref, wb_ref, bh1_ref, wh2_ref,
               bh2_ref, out_ref):
    pooled = sums_ref[...] / jnp.maximum(cnt_ref[...], 1.0)
    z1 = (jnp.dot(pooled, wa_ref[...], preferred_element_type=jnp.float32)
          + jnp.dot(meta_ref[...], wb_ref[...], preferred_element_type=jnp.float32)
          + bh1_ref[...])
    z1 = jnp.maximum(z1, 0.0)
    out_ref[...] = jnp.dot(z1, wh2_ref[...], preferred_element_type=jnp.float32) + bh2_ref[...]


def _head_call(sums, cnt, meta, wa, wb, bh1, wh2, bh2):
    return pl.pallas_call(
        _head_body,
        out_shape=jax.ShapeDtypeStruct((N_GRAPHS, 1), jnp.float32),
    )(sums, cnt, meta, wa, wb, bh1, wh2, bh2)


def kernel(x, edge_index, batch, metadata, W1, b1, W2, b2, W3, b3, Wh1, bh1,
           Wh2, bh2):
    pad = E_PAD - N_EDGES
    rowp = jnp.concatenate(
        [edge_index[0], jnp.zeros((pad,), edge_index.dtype)]).reshape(N_CHUNK_ROWS, CHUNK)
    colp = jnp.concatenate(
        [edge_index[1], jnp.full((pad,), TRASH, edge_index.dtype)]).reshape(N_CHUNK_ROWS, CHUNK)
    rowp = rowp.astype(jnp.int32)
    colp = colp.astype(jnp.int32)
    rowb = jnp.stack([rowp * 2, rowp * 2 + 1]).reshape(
        NC, N_CHUNK_ROWS // IDXR, BCHUNK)
    colp3 = colp.reshape(N_CHUNK_ROWS // IDXR, BCHUNK)
    zeros16 = jnp.zeros((ROWS_PT, 16), jnp.float32)
    zerosb = jnp.zeros((ROWS_PT, HALF), jnp.bfloat16)
    ones16 = jnp.ones((CHUNK, 16), jnp.float32)

    degp = _deg_call(colp, zeros16, ones16).reshape(NC, ACC_ROWS, 16)
    dis, g1, g1b = _enc_call(degp, x, W1)

    def agg(gb):
        out = _agg_call(gb.reshape(2 * N_NODES, HALF), rowb, colp3, zerosb)
        return out.reshape(NC, ACC_ROWS, HALF)

    b1r = b1.reshape(1, HID)
    b2r = b2.reshape(1, HID)
    b3r = b3.reshape(1, HID)

    agg1 = agg(g1b)
    g2, g2b = _layer_call(agg1, g1, dis, W2, b1r)
    agg2 = agg(g2b)
    g3, g3b = _layer_call(agg2, g2, dis, W3, b2r)
    agg3 = agg(g3b)

    sums, cnt = _pool_call(agg3, g3, dis, b3r, batch.reshape(N_NODES, 1).astype(jnp.int32))
    out = _head_call(sums, cnt, metadata, Wh1[:HID], Wh1[HID:],
                     bh1.reshape(1, HID), Wh2, bh2.reshape(1, 1))
    return out


# speedup vs baseline: 1.9393x; 1.3873x over previous
import functools

import jax
import jax.numpy as jnp
from jax import lax
from jax.experimental import pallas as pl
from jax.experimental.pallas import tpu as pltpu
from jax.experimental.pallas import tpu_sc as plsc

N_NODES = 50000
N_EDGES = 800000
IN_CH = 128
HID = 64
HALF = HID // 2
N_GRAPHS = 64

NC = 2
NS = 16
CHUNK = 128

E_PAD = 819200
N_CHUNK_ROWS = E_PAD // CHUNK
ACC_ROWS = 50176
ROWS_PT = ACC_ROWS // NS
TRASH = N_NODES

IDXR = 2
BCHUNK = IDXR * CHUNK
CPT_AGG = E_PAD // NS // BCHUNK
STG_AGG = 25
SPC_AGG = CPT_AGG // STG_AGG
CPT_DEG = E_PAD // (NC * NS) // CHUNK
STG_DEG = 5
SPC_DEG = CPT_DEG // STG_DEG

ROW_BLK = 1000
N_BLK = N_NODES // ROW_BLK

_sc_mesh = plsc.VectorSubcoreMesh(
    core_axis_name="c", subcore_axis_name="s", num_cores=NC, num_subcores=NS)


def _deg_body(col2, zeros16, ones16, out, col_st, ones_v, acc):
    c = lax.axis_index("c")
    s = lax.axis_index("s")
    pltpu.sync_copy(zeros16, acc.at[pl.ds(s * ROWS_PT, ROWS_PT)])
    pltpu.sync_copy(ones16, ones_v)
    plsc.subcore_barrier()
    base = (c * NS + s) * CPT_DEG

    def stage(st, carry):
        crb = base + st * SPC_DEG
        pltpu.sync_copy(col2.at[pl.ds(crb, SPC_DEG)], col_st)
        for j in range(SPC_DEG):
            pltpu.sync_copy(ones_v, acc.at[col_st.at[j]], add=True)
        return carry

    lax.fori_loop(0, STG_DEG, stage, 0)
    plsc.subcore_barrier()
    pltpu.sync_copy(acc.at[pl.ds(s * ROWS_PT, ROWS_PT)], out.at[c, s])


_deg_call = pl.kernel(
    _deg_body,
    out_type=jax.ShapeDtypeStruct((NC, NS, ROWS_PT, 16), jnp.float32),
    mesh=_sc_mesh,
    scratch_types=[
        pltpu.VMEM((SPC_DEG, CHUNK), jnp.int32),
        pltpu.VMEM((CHUNK, 16), jnp.float32),
        pltpu.VMEM_SHARED((ACC_ROWS, 16), jnp.float32),
    ],
    compiler_params=pltpu.CompilerParams(use_tc_tiling_on_sc=False),
)


NBUF = 4
KLAG = 2
G_RPT = N_NODES // NS


def _agg_body(gsplit, row3, col3, zerosb, out, row_st, col_st, msg, g_sh, acc,
              gsems, ssems):
    c = lax.axis_index("c")
    s = lax.axis_index("s")
    pltpu.sync_copy(zerosb, acc.at[pl.ds(s * ROWS_PT, ROWS_PT)])
    pltpu.sync_copy(gsplit.at[c, pl.ds(s * G_RPT, G_RPT)],
                    g_sh.at[pl.ds(s * G_RPT, G_RPT)])
    plsc.subcore_barrier()
    base = s * CPT_AGG

    def stage(st, carry):
        crb = base + st * SPC_AGG
        pltpu.sync_copy(row3.at[pl.ds(crb, SPC_AGG)], row_st)
        pltpu.sync_copy(col3.at[pl.ds(crb, SPC_AGG)], col_st)
        for j in range(SPC_AGG + KLAG):
            if j < SPC_AGG:
                b = j % NBUF
                if j >= NBUF:
                    pltpu.make_async_copy(msg.at[b],
                                          acc.at[col_st.at[j - NBUF]],
                                          ssems.at[b]).wait()
                pltpu.async_copy(g_sh.at[row_st.at[j]], msg.at[b], gsems.at[b])
            i = j - KLAG
            if i >= 0:
                bi = i % NBUF
                pltpu.make_async_copy(g_sh.at[row_st.at[i]], msg.at[bi],
                                      gsems.at[bi]).wait()
                pltpu.async_copy(msg.at[bi], acc.at[col_st.at[i]],
                                 ssems.at[bi], add=True)
        for k in range(NBUF):
            i = SPC_AGG - NBUF + k
            pltpu.make_async_copy(msg.at[i % NBUF], acc.at[col_st.at[i]],
                                  ssems.at[i % NBUF]).wait()
        return carry

    lax.fori_loop(0, STG_AGG, stage, 0)
    plsc.subcore_barrier()
    pltpu.sync_copy(acc.at[pl.ds(s * ROWS_PT, ROWS_PT)], out.at[c, s])


_agg_call = pl.kernel(
    _agg_body,
    out_type=jax.ShapeDtypeStruct((NC, NS, ROWS_PT, HALF), jnp.bfloat16),
    mesh=_sc_mesh,
    scratch_types=[
        pltpu.VMEM((SPC_AGG, BCHUNK), jnp.int32),
        pltpu.VMEM((SPC_AGG, BCHUNK), jnp.int32),
        pltpu.VMEM((NBUF, BCHUNK, HALF), jnp.bfloat16),
        pltpu.VMEM_SHARED((N_NODES, HALF), jnp.bfloat16),
        pltpu.VMEM_SHARED((ACC_ROWS, HALF), jnp.bfloat16),
        pltpu.SemaphoreType.DMA((NBUF,)),
        pltpu.SemaphoreType.DMA((NBUF,)),
    ],
    compiler_params=pltpu.CompilerParams(use_tc_tiling_on_sc=False),
)


def _enc_body(degp_ref, x_ref, w_ref, dis_ref, g_ref, gb_ref):
    deg = degp_ref[0, :, 0] + degp_ref[1, :, 0] + 1.0
    dis = lax.rsqrt(deg)
    dis_ref[...] = dis[:, None]
    g = jnp.dot(x_ref[...], w_ref[...], preferred_element_type=jnp.float32)
    g = g * dis[:, None]
    g_ref[...] = g
    gb_ref[0] = g[:, :HALF].astype(jnp.bfloat16)
    gb_ref[1] = g[:, HALF:].astype(jnp.bfloat16)


def _enc_call(degp, x, w1):
    return pl.pallas_call(
        _enc_body,
        grid=(N_BLK,),
        in_specs=[
            pl.BlockSpec((NC, ROW_BLK, 16), lambda i: (0, i, 0)),
            pl.BlockSpec((ROW_BLK, IN_CH), lambda i: (i, 0)),
            pl.BlockSpec((IN_CH, HID), lambda i: (0, 0)),
        ],
        out_specs=[
            pl.BlockSpec((ROW_BLK, 1), lambda i: (i, 0)),
            pl.BlockSpec((ROW_BLK, HID), lambda i: (i, 0)),
            pl.BlockSpec((NC, ROW_BLK, HALF), lambda i: (0, i, 0)),
        ],
        out_shape=[
            jax.ShapeDtypeStruct((N_NODES, 1), jnp.float32),
            jax.ShapeDtypeStruct((N_NODES, HID), jnp.float32),
            jax.ShapeDtypeStruct((NC, N_NODES, HALF), jnp.bfloat16),
        ],
    )(degp, x, w1)


def _layer_body(aggp_ref, g_ref, dis_ref, w_ref, b_ref, gn_ref, gnb_ref):
    agg = jnp.concatenate([aggp_ref[0], aggp_ref[1]],
                          axis=1).astype(jnp.float32)
    dis = dis_ref[...]
    h = jnp.maximum(dis * (agg + g_ref[...]) + b_ref[...], 0.0)
    gn = jnp.dot(h, w_ref[...], preferred_element_type=jnp.float32) * dis
    gn_ref[...] = gn
    gnb_ref[0] = gn[:, :HALF].astype(jnp.bfloat16)
    gnb_ref[1] = gn[:, HALF:].astype(jnp.bfloat16)


def _layer_call(aggp, g, dis, w, b):
    return pl.pallas_call(
        _layer_body,
        grid=(N_BLK,),
        in_specs=[
            pl.BlockSpec((NC, ROW_BLK, HALF), lambda i: (0, i, 0)),
            pl.BlockSpec((ROW_BLK, HID), lambda i: (i, 0)),
            pl.BlockSpec((ROW_BLK, 1), lambda i: (i, 0)),
            pl.BlockSpec((HID, HID), lambda i: (0, 0)),
            pl.BlockSpec((1, HID), lambda i: (0, 0)),
        ],
        out_specs=[
            pl.BlockSpec((ROW_BLK, HID), lambda i: (i, 0)),
            pl.BlockSpec((NC, ROW_BLK, HALF), lambda i: (0, i, 0)),
        ],
        out_shape=[
            jax.ShapeDtypeStruct((N_NODES, HID), jnp.float32),
            jax.ShapeDtypeStruct((NC, N_NODES, HALF), jnp.bfloat16),
        ],
    )(aggp, g, dis, w, b)


def _pool_body(aggp_ref, g_ref, dis_ref, b_ref, batch_ref, sums_ref, cnt_ref):
    i = pl.program_id(0)
    agg = jnp.concatenate([aggp_ref[0], aggp_ref[1]],
                          axis=1).astype(jnp.float32)
    dis = dis_ref[...]
    h = jnp.maximum(dis * (agg + g_ref[...]) + b_ref[...], 0.0)
    bv = batch_ref[...][:, 0]
    gids = lax.broadcasted_iota(jnp.int32, (N_GRAPHS, ROW_BLK), 0)
    m = (gids == bv[None, :]).astype(jnp.float32)
    ps = jnp.dot(m, h, preferred_element_type=jnp.float32)
    pc = jnp.sum(m, axis=1, keepdims=True)

    @pl.when(i == 0)
    def _():
        sums_ref[...] = ps
        cnt_ref[...] = pc

    @pl.when(i > 0)
    def _():
        sums_ref[...] += ps
        cnt_ref[...] += pc


def _pool_call(aggp, g, dis, b, batch2):
    return pl.pallas_call(
        _pool_body,
        grid=(N_BLK,),
        in_specs=[
            pl.BlockSpec((NC, ROW_BLK, HALF), lambda i: (0, i, 0)),
            pl.BlockSpec((ROW_BLK, HID), lambda i: (i, 0)),
            pl.BlockSpec((ROW_BLK, 1), lambda i: (i, 0)),
            pl.BlockSpec((1, HID), lambda i: (0, 0)),
            pl.BlockSpec((ROW_BLK, 1), lambda i: (i, 0)),
        ],
        out_specs=[
            pl.BlockSpec((N_GRAPHS, HID), lambda i: (0, 0)),
            pl.BlockSpec((N_GRAPHS, 1), lambda i: (0, 0)),
        ],
        out_shape=[
            jax.ShapeDtypeStruct((N_GRAPHS, HID), jnp.float32),
            jax.ShapeDtypeStruct((N_GRAPHS, 1), jnp.float32),
        ],
    )(aggp, g, dis, b, batch2)


def _head_body(sums_ref, cnt_ref, meta_ref, wa_ref, wb_ref, bh1_ref, wh2_ref,
               bh2_ref, out_ref):
    pooled = sums_ref[...] / jnp.maximum(cnt_ref[...], 1.0)
    z1 = (jnp.dot(pooled, wa_ref[...], preferred_element_type=jnp.float32)
          + jnp.dot(meta_ref[...], wb_ref[...], preferred_element_type=jnp.float32)
          + bh1_ref[...])
    z1 = jnp.maximum(z1, 0.0)
    out_ref[...] = jnp.dot(z1, wh2_ref[...], preferred_element_type=jnp.float32) + bh2_ref[...]


def _head_call(sums, cnt, meta, wa, wb, bh1, wh2, bh2):
    return pl.pallas_call(
        _head_body,
        out_shape=jax.ShapeDtypeStruct((N_GRAPHS, 1), jnp.float32),
    )(sums, cnt, meta, wa, wb, bh1, wh2, bh2)


def kernel(x, edge_index, batch, metadata, W1, b1, W2, b2, W3, b3, Wh1, bh1,
           Wh2, bh2):
    pad = E_PAD - N_EDGES
    rowp = jnp.concatenate(
        [edge_index[0], jnp.zeros((pad,), edge_index.dtype)]).reshape(N_CHUNK_ROWS, CHUNK)
    colp = jnp.concatenate(
        [edge_index[1], jnp.full((pad,), TRASH, edge_index.dtype)]).reshape(N_CHUNK_ROWS, CHUNK)
    rowp = rowp.astype(jnp.int32)
    colp = colp.astype(jnp.int32)
    rowp3 = rowp.reshape(N_CHUNK_ROWS // IDXR, BCHUNK)
    colp3 = colp.reshape(N_CHUNK_ROWS // IDXR, BCHUNK)
    zeros16 = jnp.zeros((ROWS_PT, 16), jnp.float32)
    zerosb = jnp.zeros((ROWS_PT, HALF), jnp.bfloat16)
    ones16 = jnp.ones((CHUNK, 16), jnp.float32)

    degp = _deg_call(colp, zeros16, ones16).reshape(NC, ACC_ROWS, 16)
    dis, g1, g1b = _enc_call(degp, x, W1)

    def agg(gb):
        out = _agg_call(gb, rowp3, colp3, zerosb)
        return out.reshape(NC, ACC_ROWS, HALF)

    b1r = b1.reshape(1, HID)
    b2r = b2.reshape(1, HID)
    b3r = b3.reshape(1, HID)

    agg1 = agg(g1b)
    g2, g2b = _layer_call(agg1, g1, dis, W2, b1r)
    agg2 = agg(g2b)
    g3, g3b = _layer_call(agg2, g2, dis, W3, b2r)
    agg3 = agg(g3b)

    sums, cnt = _pool_call(agg3, g3, dis, b3r, batch.reshape(N_NODES, 1).astype(jnp.int32))
    out = _head_call(sums, cnt, metadata, Wh1[:HID], Wh1[HID:],
                     bh1.reshape(1, HID), Wh2, bh2.reshape(1, 1))
    return out


# speedup vs baseline: 2.0886x; 1.0770x over previous
import jax
import jax.numpy as jnp
from jax import lax
from jax.experimental import pallas as pl
from jax.experimental.pallas import tpu as pltpu
from jax.experimental.pallas import tpu_sc as plsc

N_NODES = 50000
N_EDGES = 800000
IN_CH = 128
HID = 64
HALF = HID // 2
N_GRAPHS = 64

NC = 2
NS = 16
CHUNK = 128

E_PAD = 819200
N_CHUNK_ROWS = E_PAD // CHUNK
ACC_ROWS = 50176
ROWS_PT = ACC_ROWS // NS
TRASH = N_NODES

IDXR = 2
BCHUNK = IDXR * CHUNK
CPT_AGG = E_PAD // NS // BCHUNK
STG_AGG = 25
SPC_AGG = CPT_AGG // STG_AGG
CPT_DEG = E_PAD // (NC * NS) // CHUNK
STG_DEG = 5
SPC_DEG = CPT_DEG // STG_DEG

ROW_BLK = 2000
N_BLK = N_NODES // ROW_BLK

_sc_mesh = plsc.VectorSubcoreMesh(
    core_axis_name="c", subcore_axis_name="s", num_cores=NC, num_subcores=NS)


def _deg_body(col2, zeros16, ones16, out, col_st, ones_v, acc):
    c = lax.axis_index("c")
    s = lax.axis_index("s")
    pltpu.sync_copy(zeros16, acc.at[pl.ds(s * ROWS_PT, ROWS_PT)])
    pltpu.sync_copy(ones16, ones_v)
    plsc.subcore_barrier()
    base = (c * NS + s) * CPT_DEG

    def stage(st, carry):
        crb = base + st * SPC_DEG
        pltpu.sync_copy(col2.at[pl.ds(crb, SPC_DEG)], col_st)
        for j in range(SPC_DEG):
            pltpu.sync_copy(ones_v, acc.at[col_st.at[j]], add=True)
        return carry

    lax.fori_loop(0, STG_DEG, stage, 0)
    plsc.subcore_barrier()
    pltpu.sync_copy(acc.at[pl.ds(s * ROWS_PT, ROWS_PT)],
                    out.at[c, pl.ds(s * ROWS_PT, ROWS_PT)])


_deg_call = pl.kernel(
    _deg_body,
    out_type=jax.ShapeDtypeStruct((NC, ACC_ROWS, 16), jnp.float32),
    mesh=_sc_mesh,
    scratch_types=[
        pltpu.VMEM((SPC_DEG, CHUNK), jnp.int32),
        pltpu.VMEM((CHUNK, 16), jnp.float32),
        pltpu.VMEM_SHARED((ACC_ROWS, 16), jnp.float32),
    ],
    compiler_params=pltpu.CompilerParams(use_tc_tiling_on_sc=False),
)


NBUF = 4
KLAG = 2
G_RPT = N_NODES // NS


def _agg_body(gsplit, row3, col3, zerosb, out, row_st, col_st, msg, g_sh, acc,
              gsems, ssems):
    c = lax.axis_index("c")
    s = lax.axis_index("s")
    pltpu.sync_copy(zerosb, acc.at[pl.ds(s * ROWS_PT, ROWS_PT)])
    pltpu.sync_copy(gsplit.at[c, pl.ds(s * G_RPT, G_RPT)],
                    g_sh.at[pl.ds(s * G_RPT, G_RPT)])
    plsc.subcore_barrier()
    base = s * CPT_AGG

    def stage(st, carry):
        crb = base + st * SPC_AGG
        pltpu.sync_copy(row3.at[pl.ds(crb, SPC_AGG)], row_st)
        pltpu.sync_copy(col3.at[pl.ds(crb, SPC_AGG)], col_st)
        for j in range(SPC_AGG + KLAG):
            if j < SPC_AGG:
                b = j % NBUF
                if j >= NBUF:
                    pltpu.make_async_copy(msg.at[b],
                                          acc.at[col_st.at[j - NBUF]],
                                          ssems.at[b]).wait()
                pltpu.async_copy(g_sh.at[row_st.at[j]], msg.at[b], gsems.at[b])
            i = j - KLAG
            if i >= 0:
                bi = i % NBUF
                pltpu.make_async_copy(g_sh.at[row_st.at[i]], msg.at[bi],
                                      gsems.at[bi]).wait()
                pltpu.async_copy(msg.at[bi], acc.at[col_st.at[i]],
                                 ssems.at[bi], add=True)
        for k in range(NBUF):
            i = SPC_AGG - NBUF + k
            pltpu.make_async_copy(msg.at[i % NBUF], acc.at[col_st.at[i]],
                                  ssems.at[i % NBUF]).wait()
        return carry

    lax.fori_loop(0, STG_AGG, stage, 0)
    plsc.subcore_barrier()
    pltpu.sync_copy(acc.at[pl.ds(s * ROWS_PT, ROWS_PT)],
                    out.at[c, pl.ds(s * ROWS_PT, ROWS_PT)])


_agg_call = pl.kernel(
    _agg_body,
    out_type=jax.ShapeDtypeStruct((NC, ACC_ROWS, HALF), jnp.bfloat16),
    mesh=_sc_mesh,
    scratch_types=[
        pltpu.VMEM((SPC_AGG, BCHUNK), jnp.int32),
        pltpu.VMEM((SPC_AGG, BCHUNK), jnp.int32),
        pltpu.VMEM((NBUF, BCHUNK, HALF), jnp.bfloat16),
        pltpu.VMEM_SHARED((N_NODES, HALF), jnp.bfloat16),
        pltpu.VMEM_SHARED((ACC_ROWS, HALF), jnp.bfloat16),
        pltpu.SemaphoreType.DMA((NBUF,)),
        pltpu.SemaphoreType.DMA((NBUF,)),
    ],
    compiler_params=pltpu.CompilerParams(use_tc_tiling_on_sc=False),
)


def _enc_body(degp_ref, x_ref, w_ref, dis_ref, gb_ref):
    deg = degp_ref[0, :, 0] + degp_ref[1, :, 0] + 1.0
    dis = lax.rsqrt(deg)[:, None]
    dis_ref[...] = dis
    g = jnp.dot(x_ref[...], w_ref[...], preferred_element_type=jnp.float32)
    g = g * dis
    gb_ref[0] = g[:, :HALF].astype(jnp.bfloat16)
    gb_ref[1] = g[:, HALF:].astype(jnp.bfloat16)


def _enc_call(degp, x, w1):
    return pl.pallas_call(
        _enc_body,
        grid=(N_BLK,),
        in_specs=[
            pl.BlockSpec((NC, ROW_BLK, 16), lambda i: (0, i, 0)),
            pl.BlockSpec((ROW_BLK, IN_CH), lambda i: (i, 0)),
            pl.BlockSpec((IN_CH, HID), lambda i: (0, 0)),
        ],
        out_specs=[
            pl.BlockSpec((ROW_BLK, 1), lambda i: (i, 0)),
            pl.BlockSpec((NC, ROW_BLK, HALF), lambda i: (0, i, 0)),
        ],
        out_shape=[
            jax.ShapeDtypeStruct((N_NODES, 1), jnp.float32),
            jax.ShapeDtypeStruct((NC, N_NODES, HALF), jnp.bfloat16),
        ],
    )(degp, x, w1)


def _layer_body(aggp_ref, gb_ref, dis_ref, w_ref, b_ref, gnb_ref):
    dis = dis_ref[...]
    h0 = jnp.maximum(
        dis * (aggp_ref[0].astype(jnp.float32) + gb_ref[0].astype(jnp.float32))
        + b_ref[0], 0.0)
    h1 = jnp.maximum(
        dis * (aggp_ref[1].astype(jnp.float32) + gb_ref[1].astype(jnp.float32))
        + b_ref[1], 0.0)
    gn = (jnp.dot(h0, w_ref[:HALF, :], preferred_element_type=jnp.float32)
          + jnp.dot(h1, w_ref[HALF:, :], preferred_element_type=jnp.float32))
    gn = gn * dis
    gnb_ref[0] = gn[:, :HALF].astype(jnp.bfloat16)
    gnb_ref[1] = gn[:, HALF:].astype(jnp.bfloat16)


def _layer_call(aggp, gb, dis, w, b):
    return pl.pallas_call(
        _layer_body,
        grid=(N_BLK,),
        in_specs=[
            pl.BlockSpec((NC, ROW_BLK, HALF), lambda i: (0, i, 0)),
            pl.BlockSpec((NC, ROW_BLK, HALF), lambda i: (0, i, 0)),
            pl.BlockSpec((ROW_BLK, 1), lambda i: (i, 0)),
            pl.BlockSpec((HID, HID), lambda i: (0, 0)),
            pl.BlockSpec((NC, 1, HALF), lambda i: (0, 0, 0)),
        ],
        out_specs=pl.BlockSpec((NC, ROW_BLK, HALF), lambda i: (0, i, 0)),
        out_shape=jax.ShapeDtypeStruct((NC, N_NODES, HALF), jnp.bfloat16),
    )(aggp, gb, dis, w, b)


def _pool_body(aggp_ref, gb_ref, dis_ref, b_ref, batch_ref, sums_ref, cnt_ref):
    i = pl.program_id(0)
    dis = dis_ref[...]
    h0 = jnp.maximum(
        dis * (aggp_ref[0].astype(jnp.float32) + gb_ref[0].astype(jnp.float32))
        + b_ref[0], 0.0)
    h1 = jnp.maximum(
        dis * (aggp_ref[1].astype(jnp.float32) + gb_ref[1].astype(jnp.float32))
        + b_ref[1], 0.0)
    bv = batch_ref[...][:, 0]
    gids = lax.broadcasted_iota(jnp.int32, (N_GRAPHS, ROW_BLK), 0)
    m = (gids == bv[None, :]).astype(jnp.float32)
    s0 = jnp.dot(m, h0, preferred_element_type=jnp.float32)
    s1 = jnp.dot(m, h1, preferred_element_type=jnp.float32)
    pc = jnp.sum(m, axis=1, keepdims=True)

    @pl.when(i == 0)
    def _():
        sums_ref[0] = s0
        sums_ref[1] = s1
        cnt_ref[...] = pc

    @pl.when(i > 0)
    def _():
        sums_ref[0] += s0
        sums_ref[1] += s1
        cnt_ref[...] += pc


def _pool_call(aggp, gb, dis, b, batch2):
    return pl.pallas_call(
        _pool_body,
        grid=(N_BLK,),
        in_specs=[
            pl.BlockSpec((NC, ROW_BLK, HALF), lambda i: (0, i, 0)),
            pl.BlockSpec((NC, ROW_BLK, HALF), lambda i: (0, i, 0)),
            pl.BlockSpec((ROW_BLK, 1), lambda i: (i, 0)),
            pl.BlockSpec((NC, 1, HALF), lambda i: (0, 0, 0)),
            pl.BlockSpec((ROW_BLK, 1), lambda i: (i, 0)),
        ],
        out_specs=[
            pl.BlockSpec((NC, N_GRAPHS, HALF), lambda i: (0, 0, 0)),
            pl.BlockSpec((N_GRAPHS, 1), lambda i: (0, 0)),
        ],
        out_shape=[
            jax.ShapeDtypeStruct((NC, N_GRAPHS, HALF), jnp.float32),
            jax.ShapeDtypeStruct((N_GRAPHS, 1), jnp.float32),
        ],
    )(aggp, gb, dis, b, batch2)


def _head_body(sums_ref, cnt_ref, meta_ref, wh1_ref, wm_ref, bh1_ref, wh2_ref,
               bh2_ref, out_ref):
    inv = 1.0 / jnp.maximum(cnt_ref[...], 1.0)
    p0 = sums_ref[0] * inv
    p1 = sums_ref[1] * inv
    z1 = (jnp.dot(p0, wh1_ref[:HALF, :], preferred_element_type=jnp.float32)
          + jnp.dot(p1, wh1_ref[HALF:, :], preferred_element_type=jnp.float32)
          + jnp.dot(meta_ref[...], wm_ref[...],
                    preferred_element_type=jnp.float32)
          + bh1_ref[...])
    z1 = jnp.maximum(z1, 0.0)
    out_ref[...] = jnp.dot(z1, wh2_ref[...],
                           preferred_element_type=jnp.float32) + bh2_ref[...]


def _head_call(sums, cnt, meta, wh1a, wm, bh1, wh2, bh2):
    return pl.pallas_call(
        _head_body,
        out_shape=jax.ShapeDtypeStruct((N_GRAPHS, 1), jnp.float32),
    )(sums, cnt, meta, wh1a, wm, bh1, wh2, bh2)


def kernel(x, edge_index, batch, metadata, W1, b1, W2, b2, W3, b3, Wh1, bh1,
           Wh2, bh2):
    pad = E_PAD - N_EDGES
    rowp = jnp.concatenate(
        [edge_index[0], jnp.zeros((pad,), edge_index.dtype)]).astype(jnp.int32)
    colp = jnp.concatenate(
        [edge_index[1], jnp.full((pad,), TRASH, edge_index.dtype)]).astype(jnp.int32)
    colp2 = colp.reshape(N_CHUNK_ROWS, CHUNK)
    rowp3 = rowp.reshape(N_CHUNK_ROWS // IDXR, BCHUNK)
    colp3 = colp.reshape(N_CHUNK_ROWS // IDXR, BCHUNK)
    zeros16 = jnp.zeros((ROWS_PT, 16), jnp.float32)
    zerosb = jnp.zeros((ROWS_PT, HALF), jnp.bfloat16)
    ones16 = jnp.ones((CHUNK, 16), jnp.float32)

    degp = _deg_call(colp2, zeros16, ones16)
    dis, g1b = _enc_call(degp, x, W1)

    def agg(gb):
        return _agg_call(gb, rowp3, colp3, zerosb)

    b1r = b1.reshape(NC, 1, HALF)
    b2r = b2.reshape(NC, 1, HALF)
    b3r = b3.reshape(NC, 1, HALF)

    agg1 = agg(g1b)
    g2b = _layer_call(agg1, g1b, dis, W2, b1r)
    agg2 = agg(g2b)
    g3b = _layer_call(agg2, g2b, dis, W3, b2r)
    agg3 = agg(g3b)

    sums, cnt = _pool_call(agg3, g3b, dis, b3r,
                           batch.reshape(N_NODES, 1).astype(jnp.int32))
    out = _head_call(sums, cnt, metadata, Wh1[:HID], Wh1[HID:],
                     bh1.reshape(1, HID), Wh2, bh2.reshape(1, 1))
    return out


# speedup vs baseline: 2.1108x; 1.0107x over previous
import jax
import jax.numpy as jnp
from jax import lax
from jax.experimental import pallas as pl
from jax.experimental.pallas import tpu as pltpu
from jax.experimental.pallas import tpu_sc as plsc

N_NODES = 50000
N_EDGES = 800000
IN_CH = 128
HID = 64
HALF = HID // 2
N_GRAPHS = 64

NC = 2
NS = 16
CHUNK = 128

E_PAD = 819200
N_CHUNK_ROWS = E_PAD // CHUNK
ACC_ROWS = 50176
ROWS_PT = ACC_ROWS // NS
TRASH = N_NODES

IDXR = 2
BCHUNK = IDXR * CHUNK
CPT_AGG = E_PAD // NS // BCHUNK
STG_AGG = 25
SPC_AGG = CPT_AGG // STG_AGG
CPT_DEG = E_PAD // (NC * NS) // CHUNK
STG_DEG = 5
SPC_DEG = CPT_DEG // STG_DEG

ROW_BLK = 2000
N_BLK = N_NODES // ROW_BLK

_sc_mesh = plsc.VectorSubcoreMesh(
    core_axis_name="c", subcore_axis_name="s", num_cores=NC, num_subcores=NS)


def _deg_body(col2, zeros16, ones16, out, col_st, ones_v, acc):
    c = lax.axis_index("c")
    s = lax.axis_index("s")
    pltpu.sync_copy(zeros16, acc.at[pl.ds(s * ROWS_PT, ROWS_PT)])
    pltpu.sync_copy(ones16, ones_v)
    plsc.subcore_barrier()
    base = (c * NS + s) * CPT_DEG

    def stage(st, carry):
        crb = base + st * SPC_DEG
        pltpu.sync_copy(col2.at[pl.ds(crb, SPC_DEG)], col_st)
        for j in range(SPC_DEG):
            pltpu.sync_copy(ones_v, acc.at[col_st.at[j]], add=True)
        return carry

    lax.fori_loop(0, STG_DEG, stage, 0)
    plsc.subcore_barrier()
    pltpu.sync_copy(acc.at[pl.ds(s * ROWS_PT, ROWS_PT)],
                    out.at[c, pl.ds(s * ROWS_PT, ROWS_PT)])


_deg_call = pl.kernel(
    _deg_body,
    out_type=jax.ShapeDtypeStruct((NC, ACC_ROWS, 16), jnp.float32),
    mesh=_sc_mesh,
    scratch_types=[
        pltpu.VMEM((SPC_DEG, CHUNK), jnp.int32),
        pltpu.VMEM((CHUNK, 16), jnp.float32),
        pltpu.VMEM_SHARED((ACC_ROWS, 16), jnp.float32),
    ],
    compiler_params=pltpu.CompilerParams(use_tc_tiling_on_sc=False),
)


NBUF = 6
KLAG = 3
G_RPT = N_NODES // NS


def _agg_body(gsplit, row3, col3, zerosb, out, row_st, col_st, msg, g_sh, acc,
              gsems, ssems):
    c = lax.axis_index("c")
    s = lax.axis_index("s")
    pltpu.sync_copy(zerosb, acc.at[pl.ds(s * ROWS_PT, ROWS_PT)])
    pltpu.sync_copy(gsplit.at[c, pl.ds(s * G_RPT, G_RPT)],
                    g_sh.at[pl.ds(s * G_RPT, G_RPT)])
    plsc.subcore_barrier()
    base = s * CPT_AGG

    def stage(st, carry):
        crb = base + st * SPC_AGG
        pltpu.sync_copy(row3.at[pl.ds(crb, SPC_AGG)], row_st)
        pltpu.sync_copy(col3.at[pl.ds(crb, SPC_AGG)], col_st)
        for j in range(SPC_AGG + KLAG):
            if j < SPC_AGG:
                b = j % NBUF
                if j >= NBUF:
                    pltpu.make_async_copy(msg.at[b],
                                          acc.at[col_st.at[j - NBUF]],
                                          ssems.at[b]).wait()
                pltpu.async_copy(g_sh.at[row_st.at[j]], msg.at[b], gsems.at[b])
            i = j - KLAG
            if i >= 0:
                bi = i % NBUF
                pltpu.make_async_copy(g_sh.at[row_st.at[i]], msg.at[bi],
                                      gsems.at[bi]).wait()
                pltpu.async_copy(msg.at[bi], acc.at[col_st.at[i]],
                                 ssems.at[bi], add=True)
        for k in range(NBUF):
            i = SPC_AGG - NBUF + k
            pltpu.make_async_copy(msg.at[i % NBUF], acc.at[col_st.at[i]],
                                  ssems.at[i % NBUF]).wait()
        return carry

    lax.fori_loop(0, STG_AGG, stage, 0)
    plsc.subcore_barrier()
    pltpu.sync_copy(acc.at[pl.ds(s * ROWS_PT, ROWS_PT)],
                    out.at[c, pl.ds(s * ROWS_PT, ROWS_PT)])


_agg_call = pl.kernel(
    _agg_body,
    out_type=jax.ShapeDtypeStruct((NC, ACC_ROWS, HALF), jnp.bfloat16),
    mesh=_sc_mesh,
    scratch_types=[
        pltpu.VMEM((SPC_AGG, BCHUNK), jnp.int32),
        pltpu.VMEM((SPC_AGG, BCHUNK), jnp.int32),
        pltpu.VMEM((NBUF, BCHUNK, HALF), jnp.bfloat16),
        pltpu.VMEM_SHARED((N_NODES, HALF), jnp.bfloat16),
        pltpu.VMEM_SHARED((ACC_ROWS, HALF), jnp.bfloat16),
        pltpu.SemaphoreType.DMA((NBUF,)),
        pltpu.SemaphoreType.DMA((NBUF,)),
    ],
    compiler_params=pltpu.CompilerParams(use_tc_tiling_on_sc=False),
)


def _enc_body(degp_ref, x_ref, w_ref, dis_ref, gb_ref):
    deg = degp_ref[0, :, 0] + degp_ref[1, :, 0] + 1.0
    dis = lax.rsqrt(deg)[:, None]
    dis_ref[...] = dis
    g = jnp.dot(x_ref[...], w_ref[...], preferred_element_type=jnp.float32)
    g = g * dis
    gb_ref[0] = g[:, :HALF].astype(jnp.bfloat16)
    gb_ref[1] = g[:, HALF:].astype(jnp.bfloat16)


def _enc_call(degp, x, w1):
    return pl.pallas_call(
        _enc_body,
        grid=(N_BLK,),
        in_specs=[
            pl.BlockSpec((NC, ROW_BLK, 16), lambda i: (0, i, 0)),
            pl.BlockSpec((ROW_BLK, IN_CH), lambda i: (i, 0)),
            pl.BlockSpec((IN_CH, HID), lambda i: (0, 0)),
        ],
        out_specs=[
            pl.BlockSpec((ROW_BLK, 1), lambda i: (i, 0)),
            pl.BlockSpec((NC, ROW_BLK, HALF), lambda i: (0, i, 0)),
        ],
        out_shape=[
            jax.ShapeDtypeStruct((N_NODES, 1), jnp.float32),
            jax.ShapeDtypeStruct((NC, N_NODES, HALF), jnp.bfloat16),
        ],
    )(degp, x, w1)


def _layer_body(aggp_ref, gb_ref, dis_ref, w_ref, b_ref, gnb_ref):
    dis = dis_ref[...]
    h0 = jnp.maximum(
        dis * (aggp_ref[0].astype(jnp.float32) + gb_ref[0].astype(jnp.float32))
        + b_ref[0], 0.0)
    h1 = jnp.maximum(
        dis * (aggp_ref[1].astype(jnp.float32) + gb_ref[1].astype(jnp.float32))
        + b_ref[1], 0.0)
    gn = (jnp.dot(h0, w_ref[:HALF, :], preferred_element_type=jnp.float32)
          + jnp.dot(h1, w_ref[HALF:, :], preferred_element_type=jnp.float32))
    gn = gn * dis
    gnb_ref[0] = gn[:, :HALF].astype(jnp.bfloat16)
    gnb_ref[1] = gn[:, HALF:].astype(jnp.bfloat16)


def _layer_call(aggp, gb, dis, w, b):
    return pl.pallas_call(
        _layer_body,
        grid=(N_BLK,),
        in_specs=[
            pl.BlockSpec((NC, ROW_BLK, HALF), lambda i: (0, i, 0)),
            pl.BlockSpec((NC, ROW_BLK, HALF), lambda i: (0, i, 0)),
            pl.BlockSpec((ROW_BLK, 1), lambda i: (i, 0)),
            pl.BlockSpec((HID, HID), lambda i: (0, 0)),
            pl.BlockSpec((NC, 1, HALF), lambda i: (0, 0, 0)),
        ],
        out_specs=pl.BlockSpec((NC, ROW_BLK, HALF), lambda i: (0, i, 0)),
        out_shape=jax.ShapeDtypeStruct((NC, N_NODES, HALF), jnp.bfloat16),
    )(aggp, gb, dis, w, b)


def _pool_body(aggp_ref, gb_ref, dis_ref, b_ref, batch_ref, sums_ref, cnt_ref):
    i = pl.program_id(0)
    dis = dis_ref[...]
    h0 = jnp.maximum(
        dis * (aggp_ref[0].astype(jnp.float32) + gb_ref[0].astype(jnp.float32))
        + b_ref[0], 0.0)
    h1 = jnp.maximum(
        dis * (aggp_ref[1].astype(jnp.float32) + gb_ref[1].astype(jnp.float32))
        + b_ref[1], 0.0)
    bv = batch_ref[...][:, 0]
    gids = lax.broadcasted_iota(jnp.int32, (N_GRAPHS, ROW_BLK), 0)
    m = (gids == bv[None, :]).astype(jnp.float32)
    s0 = jnp.dot(m, h0, preferred_element_type=jnp.float32)
    s1 = jnp.dot(m, h1, preferred_element_type=jnp.float32)
    pc = jnp.sum(m, axis=1, keepdims=True)

    @pl.when(i == 0)
    def _():
        sums_ref[0] = s0
        sums_ref[1] = s1
        cnt_ref[...] = pc

    @pl.when(i > 0)
    def _():
        sums_ref[0] += s0
        sums_ref[1] += s1
        cnt_ref[...] += pc


def _pool_call(aggp, gb, dis, b, batch2):
    return pl.pallas_call(
        _pool_body,
        grid=(N_BLK,),
        in_specs=[
            pl.BlockSpec((NC, ROW_BLK, HALF), lambda i: (0, i, 0)),
            pl.BlockSpec((NC, ROW_BLK, HALF), lambda i: (0, i, 0)),
            pl.BlockSpec((ROW_BLK, 1), lambda i: (i, 0)),
            pl.BlockSpec((NC, 1, HALF), lambda i: (0, 0, 0)),
            pl.BlockSpec((ROW_BLK, 1), lambda i: (i, 0)),
        ],
        out_specs=[
            pl.BlockSpec((NC, N_GRAPHS, HALF), lambda i: (0, 0, 0)),
            pl.BlockSpec((N_GRAPHS, 1), lambda i: (0, 0)),
        ],
        out_shape=[
            jax.ShapeDtypeStruct((NC, N_GRAPHS, HALF), jnp.float32),
            jax.ShapeDtypeStruct((N_GRAPHS, 1), jnp.float32),
        ],
    )(aggp, gb, dis, b, batch2)


def _head_body(sums_ref, cnt_ref, meta_ref, wh1_ref, wm_ref, bh1_ref, wh2_ref,
               bh2_ref, out_ref):
    inv = 1.0 / jnp.maximum(cnt_ref[...], 1.0)
    p0 = sums_ref[0] * inv
    p1 = sums_ref[1] * inv
    z1 = (jnp.dot(p0, wh1_ref[:HALF, :], preferred_element_type=jnp.float32)
          + jnp.dot(p1, wh1_ref[HALF:, :], preferred_element_type=jnp.float32)
          + jnp.dot(meta_ref[...], wm_ref[...],
                    preferred_element_type=jnp.float32)
          + bh1_ref[...])
    z1 = jnp.maximum(z1, 0.0)
    out_ref[...] = jnp.dot(z1, wh2_ref[...],
                           preferred_element_type=jnp.float32) + bh2_ref[...]


def _head_call(sums, cnt, meta, wh1a, wm, bh1, wh2, bh2):
    return pl.pallas_call(
        _head_body,
        out_shape=jax.ShapeDtypeStruct((N_GRAPHS, 1), jnp.float32),
    )(sums, cnt, meta, wh1a, wm, bh1, wh2, bh2)


def kernel(x, edge_index, batch, metadata, W1, b1, W2, b2, W3, b3, Wh1, bh1,
           Wh2, bh2):
    pad = E_PAD - N_EDGES
    rowp = jnp.concatenate(
        [edge_index[0], jnp.zeros((pad,), edge_index.dtype)]).astype(jnp.int32)
    colp = jnp.concatenate(
        [edge_index[1], jnp.full((pad,), TRASH, edge_index.dtype)]).astype(jnp.int32)
    colp2 = colp.reshape(N_CHUNK_ROWS, CHUNK)
    rowp3 = rowp.reshape(N_CHUNK_ROWS // IDXR, BCHUNK)
    colp3 = colp.reshape(N_CHUNK_ROWS // IDXR, BCHUNK)
    zeros16 = jnp.zeros((ROWS_PT, 16), jnp.float32)
    zerosb = jnp.zeros((ROWS_PT, HALF), jnp.bfloat16)
    ones16 = jnp.ones((CHUNK, 16), jnp.float32)

    degp = _deg_call(colp2, zeros16, ones16)
    dis, g1b = _enc_call(degp, x, W1)

    def agg(gb):
        return _agg_call(gb, rowp3, colp3, zerosb)

    b1r = b1.reshape(NC, 1, HALF)
    b2r = b2.reshape(NC, 1, HALF)
    b3r = b3.reshape(NC, 1, HALF)

    agg1 = agg(g1b)
    g2b = _layer_call(agg1, g1b, dis, W2, b1r)
    agg2 = agg(g2b)
    g3b = _layer_call(agg2, g2b, dis, W3, b2r)
    agg3 = agg(g3b)

    sums, cnt = _pool_call(agg3, g3b, dis, b3r,
                           batch.reshape(N_NODES, 1).astype(jnp.int32))
    out = _head_call(sums, cnt, metadata, Wh1[:HID], Wh1[HID:],
                     bh1.reshape(1, HID), Wh2, bh2.reshape(1, 1))
    return out
